# Initial kernel scaffold; baseline (speedup 1.0000x reference)
import jax
import jax.numpy as jnp
from jax.experimental import pallas as pl


def kernel(time_step, class_label, cond_3d, time_table, W_proj, b_proj, class_table):
    raise NotImplementedError("write your pallas kernel here")


# speedup vs baseline: 2.1340x; 2.1340x over previous
import functools

import jax
import jax.numpy as jnp
from jax import lax
from jax.experimental import pallas as pl
from jax.experimental.pallas import tpu as pltpu
from jax.experimental.pallas import tpu_sc as plsc

B = 4
H = 224
W = 224
D = 256
NUM_CLASSES = 1000
NUM_STEPS = 1000

HWPROD = H * W
ROWS_PER_CH = HWPROD // D
CHUNK = 112
NCHUNKS = B * HWPROD // CHUNK
NLANES = 16

_SC_INFO = plsc.get_sparse_core_info()
NW = _SC_INFO.num_cores * _SC_INFO.num_subcores
CPW = NCHUNKS // NW
SGPW = B * D // NW


def _cond2d_body(ts_ref, cl_ref, tt_ref, wp_ref, bp_ref, ct_ref, out_ref):
    t_rows = jnp.concatenate(
        [tt_ref[pl.ds(ts_ref[b], 1), :] for b in range(B)], axis=0)
    c_rows = jnp.concatenate(
        [ct_ref[pl.ds(cl_ref[b], 1), :] for b in range(B)], axis=0)
    t_emb = jnp.dot(t_rows, wp_ref[...], preferred_element_type=jnp.float32)
    out_ref[...] = t_emb + bp_ref[...] + c_rows


def _cond2d(time_step, class_label, time_table, W_proj, b_proj, class_table):
    return pl.pallas_call(
        _cond2d_body,
        out_shape=jax.ShapeDtypeStruct((B, D), jnp.float32),
        in_specs=[
            pl.BlockSpec(memory_space=pltpu.SMEM),
            pl.BlockSpec(memory_space=pltpu.SMEM),
            pl.BlockSpec(memory_space=pltpu.VMEM),
            pl.BlockSpec(memory_space=pltpu.VMEM),
            pl.BlockSpec(memory_space=pltpu.VMEM),
            pl.BlockSpec(memory_space=pltpu.VMEM),
        ],
    )(time_step, class_label, time_table, W_proj,
      b_proj.reshape(1, D), class_table)


def _sc_body(table_hbm, idx_hbm, scal_hbm, out_hbm, idx_v, scal_v, buf, sem):
    wid = lax.axis_index("s") * _SC_INFO.num_cores + lax.axis_index("c")
    c0 = wid * CPW
    pltpu.sync_copy(idx_hbm.at[pl.ds(c0, CPW)], idx_v)
    pltpu.sync_copy(scal_hbm.at[pl.ds(wid * SGPW, SGPW)], scal_v)

    def chunk_body(j, carry):
        lo, bnd = carry
        pltpu.async_copy(table_hbm.at[idx_v.at[j]], buf, sem).wait()
        v_lo = scal_v[lo]
        v_hi = scal_v[jnp.minimum(lo + 1, SGPW - 1)]

        def row_body(r, c2):
            svec = jnp.where(r < bnd, v_lo, v_hi)
            for cc in range(D // NLANES):
                sl = pl.ds(cc * NLANES, NLANES)
                buf[r, sl] = buf[r, sl] + svec
            return c2

        lax.fori_loop(0, CHUNK, row_body, 0)
        pltpu.sync_copy(buf, out_hbm.at[c0 + j])
        crossed = bnd <= CHUNK
        nlo = lo + jnp.where(crossed, 1, 0)
        nbnd = bnd - CHUNK + jnp.where(crossed, ROWS_PER_CH, 0)
        return (nlo, nbnd)

    lax.fori_loop(0, CPW, chunk_body,
                  (jnp.int32(0), jnp.int32(ROWS_PER_CH)))


@functools.partial(
    pl.kernel,
    mesh=plsc.VectorSubcoreMesh(core_axis_name="c", subcore_axis_name="s"),
    out_type=jax.ShapeDtypeStruct((NCHUNKS, CHUNK, D), jnp.float32),
    scratch_types=[
        pltpu.VMEM((CPW, CHUNK), jnp.int32),
        pltpu.VMEM((SGPW, NLANES), jnp.float32),
        pltpu.VMEM((CHUNK, D), jnp.float32),
        pltpu.SemaphoreType.DMA,
    ],
)
def _sc_gather_add(table_hbm, idx_hbm, scal_hbm, out_hbm,
                   idx_v, scal_v, buf, sem):
    _sc_body(table_hbm, idx_hbm, scal_hbm, out_hbm, idx_v, scal_v, buf, sem)


def kernel(time_step, class_label, cond_3d, time_table, W_proj, b_proj,
           class_table):
    cond2d = _cond2d(time_step, class_label, time_table, W_proj, b_proj,
                     class_table)
    scal = jnp.broadcast_to(
        cond2d.reshape(B * D, 1), (B * D, NLANES))
    idx = cond_3d.reshape(NCHUNKS, CHUNK)
    out = _sc_gather_add(class_table, idx, scal)
    return out.reshape(B, D, H, W)
